# baseline (speedup 1.0000x reference)
import jax
import jax.numpy as jnp
from jax import lax
from jax.experimental import pallas as pl
from jax.experimental.pallas import tpu as pltpu
from jax.experimental.pallas import tpu_sc as plsc

NSVC, NPOD, NNODE = 10000, 50000, 10000
ESVC, EIN, ENI = 320000, 50000, 50000
NC, NS, K = 2, 16, 128

EP_SVC = 80 * 32 * K
EP_IN = 14 * 32 * K
EP_NI = 26 * 16 * K

DEG_SIZES = (10008, 10008, 50008, 10008, 10008, 50008)
DEG_CPT = (80, 80, 14, 14, 14, 14)

SP1 = 10112
SPNI = 12928


def _mesh():
    return plsc.VectorSubcoreMesh(core_axis_name="c", subcore_axis_name="s",
                                  num_cores=NC, num_subcores=NS)


DEG_TOT = 140048
DEG_OFF = (0, 10008, 20016, 70024, 80032, 90040)


def _deg_body(*args):
    idx_in = args[0:6]
    ones_hbm, z_all = args[6], args[7]
    out0, out1 = args[8], args[9]
    ones_v = args[10]
    stages = args[11:17]
    g0, g1 = args[17], args[18]
    shared = args[19]
    c = lax.axis_index("c")
    s = lax.axis_index("s")
    t = c * NS + s
    pltpu.sync_copy(ones_hbm, ones_v)

    @pl.when(s == 0)
    def _():
        pltpu.sync_copy(z_all, shared)
    plsc.subcore_barrier()
    for a in range(6):
        cpt = DEG_CPT[a]
        pltpu.sync_copy(idx_in[a].at[t], stages[a])

        def body(i, _, a=a, cpt=cpt):
            d0 = pltpu.async_copy(ones_v, shared.at[stages[a].at[2 * i]],
                                  g0, add=True)
            d1 = pltpu.async_copy(ones_v, shared.at[stages[a].at[2 * i + 1]],
                                  g1, add=True)
            d0.wait()
            d1.wait()
            return 0

        lax.fori_loop(0, cpt // 2, body, 0)
    plsc.subcore_barrier()

    @pl.when(s == 0)
    def _():
        @pl.when(c == 0)
        def _():
            pltpu.sync_copy(shared, out0)

        @pl.when(c == 1)
        def _():
            pltpu.sync_copy(shared, out1)


def _deg_call(*idx_arrays):
    ones128 = jnp.ones((K,), jnp.float32)
    z_all = jnp.zeros((DEG_TOT,), jnp.float32)
    f = pl.kernel(
        _deg_body,
        out_type=(jax.ShapeDtypeStruct((DEG_TOT,), jnp.float32),
                  jax.ShapeDtypeStruct((DEG_TOT,), jnp.float32)),
        mesh=_mesh(),
        compiler_params=pltpu.CompilerParams(needs_layout_passes=False),
        scratch_types=(
            [pltpu.VMEM((K,), jnp.float32)]
            + [pltpu.VMEM((cpt, K), jnp.int32) for cpt in DEG_CPT]
            + [pltpu.SemaphoreType.DMA, pltpu.SemaphoreType.DMA]
            + [pltpu.VMEM_SHARED((DEG_TOT,), jnp.float32)]
        ),
    )
    return f(*idx_arrays, ones128, z_all)


def _matmul(x, d0, d1, w):
    n = x.shape[0]
    nb = n // 400

    def body(x_ref, d0_ref, d1_ref, w_ref, o_ref):
        deg = d0_ref[...] + d1_ref[...]
        norm = lax.rsqrt(jnp.maximum(deg, 1.0))
        o_ref[...] = jnp.dot(x_ref[...] * norm, w_ref[...],
                             preferred_element_type=jnp.float32)

    return pl.pallas_call(
        body,
        grid=(nb,),
        in_specs=[
            pl.BlockSpec((400, 128), lambda i: (i, 0)),
            pl.BlockSpec((400, 1), lambda i: (i, 0)),
            pl.BlockSpec((400, 1), lambda i: (i, 0)),
            pl.BlockSpec((128, 128), lambda i: (0, 0)),
        ],
        out_specs=pl.BlockSpec((400, 128), lambda i: (i, 0)),
        out_shape=jax.ShapeDtypeStruct((n, 128), jnp.float32),
    )(x, d0, d1, w)


SPB = 5632


def _aggall_body(h_svc, h_in, h_ni, svcS1, svcD1, inS1, inD1, niS1, niD1,
                 zeros2d, fillD, o_svc, o_in, o_ni,
                 sSf, sDf, iSf, iDf, nSf, nDf, SlF, DlF, ridx0, ridx1,
                 rows, g0, g1, s0, s1, agg):
    c = lax.axis_index("c")
    s = lax.axis_index("s")
    t = c * NS + s
    gsem = (g0, g1)
    ssem = (s0, s1)
    ridx = (ridx0, ridx1)

    def zero():
        pltpu.sync_copy(zeros2d.at[pl.ds(0, 352)],
                        agg.at[pl.ds(s * 352, 352)])

    def compact(srcf, dstf, nv, base):
        pltpu.sync_copy(fillD.at[pl.ds(0, nv * 16)], DlF.at[pl.ds(0, nv * 16)])
        pltpu.sync_copy(fillD.at[pl.ds(0, nv * 16)], SlF.at[pl.ds(0, nv * 16)])

        def it(vi, cnt):
            d = dstf[pl.ds(vi * 16, 16)]
            sv = srcf[pl.ds(vi * 16, 16)]
            loc = d - base
            m = (loc >= 0) & (loc < 5600)
            pref = plsc.cumsum(m.astype(jnp.int32))
            pos = cnt + pref - 1
            plsc.store_scatter(DlF, [pos], loc, mask=m)
            plsc.store_scatter(SlF, [pos], sv, mask=m)
            return cnt + jnp.max(pref)

        cnt = lax.fori_loop(0, nv, it, jnp.int32(0))
        return (cnt + 255) // 256

    def pump(h, nc2):
        def gstart(k, u):
            pltpu.async_copy(h.at[SlF.at[pl.ds(k * K, K)]], rows.at[u],
                             gsem[u])

        def gwait(k, u):
            pltpu.make_async_copy(h.at[SlF.at[pl.ds(k * K, K)]], rows.at[u],
                                  gsem[u]).wait()

        def sstart(k, u):
            pltpu.async_copy(rows.at[u], agg.at[ridx[u]], ssem[u], add=True)

        def swait(u):
            pltpu.make_async_copy(rows.at[u], agg.at[ridx[u]],
                                  ssem[u]).wait()

        @pl.when(nc2 > 0)
        def _():
            gstart(0, 0)

            def it(i, _):
                for u in (0, 1):
                    k = 2 * i + u
                    gwait(k, u)
                    for j in range(8):
                        ridx[u][pl.ds(j * 16, 16)] = (
                            DlF[pl.ds(k * K + j * 16, 16)])
                    if u == 0:
                        @pl.when(i > 0)
                        def _():
                            swait(1)
                    else:
                        swait(0)
                    nk = lax.rem(k + 1, 2 * nc2)
                    gstart(nk, 1 - u)
                    sstart(k, u)
                return 0

            lax.fori_loop(0, nc2, it, 0)
            swait(1)
            gwait(0, 0)

    def dump(out_hbm, row0):
        @pl.when(s < 10)
        def _():
            pltpu.sync_copy(agg.at[pl.ds(s * 560, 560)],
                            out_hbm.at[pl.ds(row0 + s * 560, 560)])

    pltpu.sync_copy(svcS1.at[pl.ds(t * 10240, 10240)], sSf)
    pltpu.sync_copy(svcD1.at[pl.ds(t * 10240, 10240)], sDf)
    pltpu.sync_copy(inS1.at[pl.ds(t * 1792, 1792)], iSf)
    pltpu.sync_copy(inD1.at[pl.ds(t * 1792, 1792)], iDf)
    pltpu.sync_copy(niS1.at[pl.ds(s * 3328, 3328)], nSf)
    pltpu.sync_copy(niD1.at[pl.ds(s * 3328, 3328)], nDf)

    for q in (0, 1):
        zero()
        nc2 = compact(sSf, sDf, 640, q * 5600)
        plsc.subcore_barrier()
        pump(h_svc, nc2)
        plsc.subcore_barrier()
        dump(o_svc, (2 * c + q) * 5600)
        plsc.subcore_barrier()
        zero()
        nc2 = compact(iSf, iDf, 112, q * 5600)
        plsc.subcore_barrier()
        pump(h_in, nc2)
        plsc.subcore_barrier()
        dump(o_in, (2 * c + q) * 5600)
        plsc.subcore_barrier()

    for q in (0, 1, 2, 3, 4):
        zero()
        nc2 = compact(nSf, nDf, 208, (2 * q + c) * 5600)
        plsc.subcore_barrier()
        pump(h_ni, nc2)
        plsc.subcore_barrier()
        dump(o_ni, (2 * q + c) * 5600)
        plsc.subcore_barrier()


def _aggall_call(h_svc, h_in, h_ni, svcS1, svcD1, inS1, inD1, niS1, niD1):
    zeros2d = jnp.zeros((440, 128), jnp.float32)
    fillD = jnp.full((10240,), 5631, jnp.int32)
    f = pl.kernel(
        _aggall_body,
        out_type=(jax.ShapeDtypeStruct((22400, 128), jnp.float32),
                  jax.ShapeDtypeStruct((22400, 128), jnp.float32),
                  jax.ShapeDtypeStruct((56000, 128), jnp.float32)),
        mesh=_mesh(),
        compiler_params=pltpu.CompilerParams(needs_layout_passes=False),
        scratch_types=(
            pltpu.VMEM((10240,), jnp.int32),
            pltpu.VMEM((10240,), jnp.int32),
            pltpu.VMEM((1792,), jnp.int32),
            pltpu.VMEM((1792,), jnp.int32),
            pltpu.VMEM((3328,), jnp.int32),
            pltpu.VMEM((3328,), jnp.int32),
            pltpu.VMEM((10240,), jnp.int32),
            pltpu.VMEM((10240,), jnp.int32),
            pltpu.VMEM((K,), jnp.int32),
            pltpu.VMEM((K,), jnp.int32),
            pltpu.VMEM((2, K, 128), jnp.float32),
            pltpu.SemaphoreType.DMA,
            pltpu.SemaphoreType.DMA,
            pltpu.SemaphoreType.DMA,
            pltpu.SemaphoreType.DMA,
            pltpu.VMEM_SHARED((SPB, 128), jnp.float32),
        ),
    )
    return f(h_svc, h_in, h_ni, svcS1, svcD1, inS1, inD1, niS1, niD1,
             zeros2d, fillD)


def _reduce(parts, d0, d1, b2d, wl, bl2d, n):
    nb = n // 400
    nparts = len(parts)

    def body(*refs):
        a_refs = refs[:nparts]
        d0_ref, d1_ref, b_ref, wl_ref, bl_ref, o_ref, acc = refs[nparts:]
        i = pl.program_id(0)
        h = a_refs[0][...].astype(jnp.float32)
        for r in a_refs[1:]:
            h = h + r[...].astype(jnp.float32)
        deg = d0_ref[...] + d1_ref[...]
        norm = lax.rsqrt(jnp.maximum(deg, 1.0))
        h = h * norm + b_ref[...]
        lh = jnp.where(h > 0, h, 0.01 * h)
        cs = jnp.sum(lh, axis=0, keepdims=True)

        @pl.when(i == 0)
        def _():
            acc[...] = cs

        @pl.when(i > 0)
        def _():
            acc[...] = acc[...] + cs

        @pl.when(i == nb - 1)
        def _():
            wvec = jnp.sum(wl_ref[...], axis=0, keepdims=True)
            o_ref[...] = (jnp.sum(acc[...] * wvec, keepdims=True)
                          + n * jnp.sum(bl_ref[...], keepdims=True))

    in_specs = (
        [pl.BlockSpec((400, 128), lambda i, o=off: (i + o, 0))
         for off in [p[1] for p in parts]]
        + [pl.BlockSpec((400, 1), lambda i: (i, 0)),
           pl.BlockSpec((400, 1), lambda i: (i, 0)),
           pl.BlockSpec((1, 128), lambda i: (0, 0)),
           pl.BlockSpec((128, 128), lambda i: (0, 0)),
           pl.BlockSpec((1, 128), lambda i: (0, 0))]
    )
    return pl.pallas_call(
        body,
        grid=(nb,),
        in_specs=in_specs,
        out_specs=pl.BlockSpec((1, 1), lambda i: (0, 0)),
        out_shape=jax.ShapeDtypeStruct((1, 1), jnp.float32),
        scratch_shapes=[pltpu.VMEM((1, 128), jnp.float32)],
    )(*[p[0] for p in parts], d0, d1, b2d, wl, bl2d)


def _pad3d(x, epad, fill, ntiles):
    pad = jnp.full((epad - x.shape[0],), fill, jnp.int32)
    cpt = epad // (ntiles * K)
    return jnp.concatenate([x.astype(jnp.int32), pad]).reshape(ntiles, cpt, K)


def _pad1d(x, epad, fill):
    pad = jnp.full((epad - x.shape[0],), fill, jnp.int32)
    return jnp.concatenate([x.astype(jnp.int32), pad])


def _degslice(d, off, n):
    return d[off:off + n].reshape(n, 1)


def kernel(feat_svc, feat_pod, feat_node, svc_edges, in_src, in_dst,
           ni_src, ni_dst, W_svc, b_svc, W_in, b_in, W_ni, b_ni,
           Wl_svc, bl_svc, Wl_pod, bl_pod, Wl_node, bl_node):
    svc_s = svc_edges[0]
    svc_d = svc_edges[1]
    svcS1 = _pad1d(svc_s, EP_SVC, 0)
    svcD1 = _pad1d(svc_d, EP_SVC, 1 << 20)
    inS1 = _pad1d(in_src, EP_IN, 0)
    inD1 = _pad1d(in_dst, EP_IN, 1 << 20)
    niS1 = _pad1d(ni_src, EP_NI, 0)
    niD1 = _pad1d(ni_dst, EP_NI, 1 << 20)
    svcS2d = _pad3d(svc_s + DEG_OFF[0], EP_SVC, DEG_OFF[0] + 10000, 32)
    svcD2d = _pad3d(svc_d + DEG_OFF[1], EP_SVC, DEG_OFF[1] + 10000, 32)
    inS2d = _pad3d(in_src + DEG_OFF[2], EP_IN, DEG_OFF[2] + 50000, 32)
    inD2d = _pad3d(in_dst + DEG_OFF[3], EP_IN, DEG_OFF[3] + 10000, 32)
    niS2d = _pad3d(ni_src + DEG_OFF[4], EP_IN, DEG_OFF[4] + 10000, 32)
    niD2d = _pad3d(ni_dst + DEG_OFF[5], EP_IN, DEG_OFF[5] + 50000, 32)

    d0, d1 = _deg_call(svcS2d, svcD2d, inS2d, inD2d, niS2d, niD2d)

    h_svc = _matmul(feat_svc, _degslice(d0, DEG_OFF[0], NSVC),
                    _degslice(d1, DEG_OFF[0], NSVC), W_svc)
    h_in = _matmul(feat_pod, _degslice(d0, DEG_OFF[2], NPOD),
                   _degslice(d1, DEG_OFF[2], NPOD), W_in)
    h_ni = _matmul(feat_node, _degslice(d0, DEG_OFF[4], NNODE),
                   _degslice(d1, DEG_OFF[4], NNODE), W_ni)

    agg_svc, agg_in, agg_ni = _aggall_call(
        h_svc, h_in, h_ni, svcS1, svcD1, inS1, inD1, niS1, niD1)

    s1 = _reduce([(agg_svc, 0), (agg_svc, 28)],
                 _degslice(d0, DEG_OFF[1], NSVC), _degslice(d1, DEG_OFF[1], NSVC),
                 b_svc.reshape(1, 128), Wl_svc, bl_svc.reshape(1, 128),
                 NSVC)
    s2 = _reduce([(agg_in, 0), (agg_in, 28)],
                 _degslice(d0, DEG_OFF[3], NNODE), _degslice(d1, DEG_OFF[3], NNODE),
                 b_in.reshape(1, 128), Wl_node, bl_node.reshape(1, 128),
                 NNODE)
    s3 = _reduce([(agg_ni, 0)],
                 _degslice(d0, DEG_OFF[5], NPOD), _degslice(d1, DEG_OFF[5], NPOD),
                 b_ni.reshape(1, 128), Wl_pod, bl_pod.reshape(1, 128),
                 NPOD)
    total = (s1[0, 0] + s2[0, 0] + s3[0, 0]) / float((NSVC + NPOD + NNODE)
                                                     * 128)
    return total


# speedup vs baseline: 1.0007x; 1.0007x over previous
import jax
import jax.numpy as jnp
from jax import lax
from jax.experimental import pallas as pl
from jax.experimental.pallas import tpu as pltpu
from jax.experimental.pallas import tpu_sc as plsc

NSVC, NPOD, NNODE = 10000, 50000, 10000
ESVC, EIN, ENI = 320000, 50000, 50000
NC, NS, K = 2, 16, 128

EP_SVC = 80 * 32 * K
EP_IN = 14 * 32 * K
EP_NI = 26 * 16 * K

DEG_SIZES = (10008, 10008, 50008, 10008, 10008, 50008)
DEG_CPT = (80, 80, 14, 14, 14, 14)

def _mesh():
    return plsc.VectorSubcoreMesh(core_axis_name="c", subcore_axis_name="s",
                                  num_cores=NC, num_subcores=NS)


DEG_TOT = 140048
DEG_OFF = (0, 10008, 20016, 70024, 80032, 90040)


def _deg_body(*args):
    idx_in = args[0:6]
    ones_hbm, z_all = args[6], args[7]
    out0, out1 = args[8], args[9]
    ones_v = args[10]
    stages = args[11:17]
    g0, g1 = args[17], args[18]
    shared = args[19]
    c = lax.axis_index("c")
    s = lax.axis_index("s")
    t = c * NS + s
    pltpu.sync_copy(ones_hbm, ones_v)

    @pl.when(s == 0)
    def _():
        pltpu.sync_copy(z_all, shared)
    plsc.subcore_barrier()
    for a in range(6):
        cpt = DEG_CPT[a]
        pltpu.sync_copy(idx_in[a].at[t], stages[a])

        def body(i, _, a=a, cpt=cpt):
            d0 = pltpu.async_copy(ones_v, shared.at[stages[a].at[2 * i]],
                                  g0, add=True)
            d1 = pltpu.async_copy(ones_v, shared.at[stages[a].at[2 * i + 1]],
                                  g1, add=True)
            d0.wait()
            d1.wait()
            return 0

        lax.fori_loop(0, cpt // 2, body, 0)
    plsc.subcore_barrier()

    @pl.when(s == 0)
    def _():
        @pl.when(c == 0)
        def _():
            pltpu.sync_copy(shared, out0)

        @pl.when(c == 1)
        def _():
            pltpu.sync_copy(shared, out1)


def _deg_call(*idx_arrays):
    ones128 = jnp.ones((K,), jnp.float32)
    z_all = jnp.zeros((DEG_TOT,), jnp.float32)
    f = pl.kernel(
        _deg_body,
        out_type=(jax.ShapeDtypeStruct((DEG_TOT,), jnp.float32),
                  jax.ShapeDtypeStruct((DEG_TOT,), jnp.float32)),
        mesh=_mesh(),
        compiler_params=pltpu.CompilerParams(needs_layout_passes=False),
        scratch_types=(
            [pltpu.VMEM((K,), jnp.float32)]
            + [pltpu.VMEM((cpt, K), jnp.int32) for cpt in DEG_CPT]
            + [pltpu.SemaphoreType.DMA, pltpu.SemaphoreType.DMA]
            + [pltpu.VMEM_SHARED((DEG_TOT,), jnp.float32)]
        ),
    )
    return f(*idx_arrays, ones128, z_all)


def _matmul(x, d0, d1, w):
    n = x.shape[0]
    nb = n // 400

    def body(x_ref, d0_ref, d1_ref, w_ref, o_ref):
        deg = d0_ref[...] + d1_ref[...]
        norm = lax.rsqrt(jnp.maximum(deg, 1.0))
        o_ref[...] = jnp.dot(x_ref[...] * norm, w_ref[...],
                             preferred_element_type=jnp.float32)

    return pl.pallas_call(
        body,
        grid=(nb,),
        in_specs=[
            pl.BlockSpec((400, 128), lambda i: (i, 0)),
            pl.BlockSpec((400, 1), lambda i: (i, 0)),
            pl.BlockSpec((400, 1), lambda i: (i, 0)),
            pl.BlockSpec((128, 128), lambda i: (0, 0)),
        ],
        out_specs=pl.BlockSpec((400, 128), lambda i: (i, 0)),
        out_shape=jax.ShapeDtypeStruct((n, 128), jnp.float32),
    )(x, d0, d1, w)


SPB = 5632


def _aggall_body(h_svc, h_in, h_ni, svcS1, svcD1, inS1, inD1, niS1, niD1,
                 zeros2d, fillD, o_svc, o_in, o_ni,
                 sSf, sDf, iSf, iDf, nSf, nDf, SlF, DlF, ridx0, ridx1,
                 rows, g0, g1, s0, s1, agg):
    c = lax.axis_index("c")
    s = lax.axis_index("s")
    t = c * NS + s
    gsem = (g0, g1)
    ssem = (s0, s1)
    ridx = (ridx0, ridx1)

    def zero():
        pltpu.sync_copy(zeros2d.at[pl.ds(0, 352)],
                        agg.at[pl.ds(s * 352, 352)])

    def compact(srcf, dstf, nv, base):
        pltpu.sync_copy(fillD.at[pl.ds(0, nv * 16)], DlF.at[pl.ds(0, nv * 16)])
        pltpu.sync_copy(fillD.at[pl.ds(0, nv * 16)], SlF.at[pl.ds(0, nv * 16)])

        def it(vi, cnt):
            d = dstf[pl.ds(vi * 16, 16)]
            sv = srcf[pl.ds(vi * 16, 16)]
            loc = d - base
            m = (loc >= 0) & (loc < 5600)
            pref = plsc.cumsum(m.astype(jnp.int32))
            pos = cnt + pref - 1
            plsc.store_scatter(DlF, [pos], loc, mask=m)
            plsc.store_scatter(SlF, [pos], sv, mask=m)
            return cnt + jnp.max(pref)

        cnt = lax.fori_loop(0, nv, it, jnp.int32(0))
        return (cnt + 255) // 256

    def pump(h, nc2):
        def gstart(k, u):
            pltpu.async_copy(h.at[SlF.at[pl.ds(k * K, K)]], rows.at[u],
                             gsem[u])

        def gwait(k, u):
            pltpu.make_async_copy(h.at[SlF.at[pl.ds(k * K, K)]], rows.at[u],
                                  gsem[u]).wait()

        def sstart(k, u):
            pltpu.async_copy(rows.at[u], agg.at[ridx[u]], ssem[u], add=True)

        def swait(u):
            pltpu.make_async_copy(rows.at[u], agg.at[ridx[u]],
                                  ssem[u]).wait()

        @pl.when(nc2 > 0)
        def _():
            gstart(0, 0)

            def it(i, _):
                for u in (0, 1):
                    k = 2 * i + u
                    gwait(k, u)
                    for j in range(8):
                        ridx[u][pl.ds(j * 16, 16)] = (
                            DlF[pl.ds(k * K + j * 16, 16)])
                    if u == 0:
                        @pl.when(i > 0)
                        def _():
                            swait(1)
                    else:
                        swait(0)
                    nk = lax.rem(k + 1, 2 * nc2)
                    gstart(nk, 1 - u)
                    sstart(k, u)
                return 0

            lax.fori_loop(0, nc2, it, 0)
            swait(1)
            gwait(0, 0)

    def dump(out_hbm, row0):
        @pl.when(s < 10)
        def _():
            pltpu.sync_copy(agg.at[pl.ds(s * 560, 560)],
                            out_hbm.at[pl.ds(row0 + s * 560, 560)])

    pltpu.sync_copy(svcS1.at[pl.ds(t * 10240, 10240)], sSf)
    pltpu.sync_copy(svcD1.at[pl.ds(t * 10240, 10240)], sDf)
    pltpu.sync_copy(inS1.at[pl.ds(t * 1792, 1792)], iSf)
    pltpu.sync_copy(inD1.at[pl.ds(t * 1792, 1792)], iDf)
    pltpu.sync_copy(niS1.at[pl.ds(s * 3328, 3328)], nSf)
    pltpu.sync_copy(niD1.at[pl.ds(s * 3328, 3328)], nDf)

    for q in (0, 1):
        zero()
        nc2 = compact(sSf, sDf, 640, q * 5600)
        plsc.subcore_barrier()
        pump(h_svc, nc2)
        plsc.subcore_barrier()
        dump(o_svc, (2 * c + q) * 5600)
        plsc.subcore_barrier()
        zero()
        nc2 = compact(iSf, iDf, 112, q * 5600)
        plsc.subcore_barrier()
        pump(h_in, nc2)
        plsc.subcore_barrier()
        dump(o_in, (2 * c + q) * 5600)
        plsc.subcore_barrier()

    for q in (0, 1, 2, 3, 4):
        zero()
        nc2 = compact(nSf, nDf, 208, (2 * q + c) * 5600)
        plsc.subcore_barrier()
        pump(h_ni, nc2)
        plsc.subcore_barrier()
        dump(o_ni, (2 * q + c) * 5600)
        plsc.subcore_barrier()


def _aggall_call(h_svc, h_in, h_ni, svcS1, svcD1, inS1, inD1, niS1, niD1):
    zeros2d = jnp.zeros((440, 128), jnp.float32)
    fillD = jnp.full((10240,), 5631, jnp.int32)
    f = pl.kernel(
        _aggall_body,
        out_type=(jax.ShapeDtypeStruct((22400, 128), jnp.float32),
                  jax.ShapeDtypeStruct((22400, 128), jnp.float32),
                  jax.ShapeDtypeStruct((56000, 128), jnp.float32)),
        mesh=_mesh(),
        compiler_params=pltpu.CompilerParams(needs_layout_passes=False),
        scratch_types=(
            pltpu.VMEM((10240,), jnp.int32),
            pltpu.VMEM((10240,), jnp.int32),
            pltpu.VMEM((1792,), jnp.int32),
            pltpu.VMEM((1792,), jnp.int32),
            pltpu.VMEM((3328,), jnp.int32),
            pltpu.VMEM((3328,), jnp.int32),
            pltpu.VMEM((10240,), jnp.int32),
            pltpu.VMEM((10240,), jnp.int32),
            pltpu.VMEM((K,), jnp.int32),
            pltpu.VMEM((K,), jnp.int32),
            pltpu.VMEM((2, K, 128), jnp.float32),
            pltpu.SemaphoreType.DMA,
            pltpu.SemaphoreType.DMA,
            pltpu.SemaphoreType.DMA,
            pltpu.SemaphoreType.DMA,
            pltpu.VMEM_SHARED((SPB, 128), jnp.float32),
        ),
    )
    return f(h_svc, h_in, h_ni, svcS1, svcD1, inS1, inD1, niS1, niD1,
             zeros2d, fillD)


def _reduce(parts, d0, d1, b2d, wl, bl2d, n):
    nb = n // 400
    nparts = len(parts)

    def body(*refs):
        a_refs = refs[:nparts]
        d0_ref, d1_ref, b_ref, wl_ref, bl_ref, o_ref, acc = refs[nparts:]
        i = pl.program_id(0)
        h = a_refs[0][...].astype(jnp.float32)
        for r in a_refs[1:]:
            h = h + r[...].astype(jnp.float32)
        deg = d0_ref[...] + d1_ref[...]
        norm = lax.rsqrt(jnp.maximum(deg, 1.0))
        h = h * norm + b_ref[...]
        lh = jnp.where(h > 0, h, 0.01 * h)
        cs = jnp.sum(lh, axis=0, keepdims=True)

        @pl.when(i == 0)
        def _():
            acc[...] = cs

        @pl.when(i > 0)
        def _():
            acc[...] = acc[...] + cs

        @pl.when(i == nb - 1)
        def _():
            wvec = jnp.sum(wl_ref[...], axis=0, keepdims=True)
            o_ref[...] = (jnp.sum(acc[...] * wvec, keepdims=True)
                          + n * jnp.sum(bl_ref[...], keepdims=True))

    in_specs = (
        [pl.BlockSpec((400, 128), lambda i, o=off: (i + o, 0))
         for off in [p[1] for p in parts]]
        + [pl.BlockSpec((400, 1), lambda i: (i, 0)),
           pl.BlockSpec((400, 1), lambda i: (i, 0)),
           pl.BlockSpec((1, 128), lambda i: (0, 0)),
           pl.BlockSpec((128, 128), lambda i: (0, 0)),
           pl.BlockSpec((1, 128), lambda i: (0, 0))]
    )
    return pl.pallas_call(
        body,
        grid=(nb,),
        in_specs=in_specs,
        out_specs=pl.BlockSpec((1, 1), lambda i: (0, 0)),
        out_shape=jax.ShapeDtypeStruct((1, 1), jnp.float32),
        scratch_shapes=[pltpu.VMEM((1, 128), jnp.float32)],
    )(*[p[0] for p in parts], d0, d1, b2d, wl, bl2d)


def _pad3d(x, epad, fill, ntiles):
    pad = jnp.full((epad - x.shape[0],), fill, jnp.int32)
    cpt = epad // (ntiles * K)
    return jnp.concatenate([x.astype(jnp.int32), pad]).reshape(ntiles, cpt, K)


def _pad1d(x, epad, fill):
    pad = jnp.full((epad - x.shape[0],), fill, jnp.int32)
    return jnp.concatenate([x.astype(jnp.int32), pad])


def _degslice(d, off, n):
    return d[off:off + n].reshape(n, 1)


def kernel(feat_svc, feat_pod, feat_node, svc_edges, in_src, in_dst,
           ni_src, ni_dst, W_svc, b_svc, W_in, b_in, W_ni, b_ni,
           Wl_svc, bl_svc, Wl_pod, bl_pod, Wl_node, bl_node):
    svc_s = svc_edges[0]
    svc_d = svc_edges[1]
    svcS1 = _pad1d(svc_s, EP_SVC, 0)
    svcD1 = _pad1d(svc_d, EP_SVC, 1 << 20)
    inS1 = _pad1d(in_src, EP_IN, 0)
    inD1 = _pad1d(in_dst, EP_IN, 1 << 20)
    niS1 = _pad1d(ni_src, EP_NI, 0)
    niD1 = _pad1d(ni_dst, EP_NI, 1 << 20)
    svcS2d = _pad3d(svc_s + DEG_OFF[0], EP_SVC, DEG_OFF[0] + 10000, 32)
    svcD2d = _pad3d(svc_d + DEG_OFF[1], EP_SVC, DEG_OFF[1] + 10000, 32)
    inS2d = _pad3d(in_src + DEG_OFF[2], EP_IN, DEG_OFF[2] + 50000, 32)
    inD2d = _pad3d(in_dst + DEG_OFF[3], EP_IN, DEG_OFF[3] + 10000, 32)
    niS2d = _pad3d(ni_src + DEG_OFF[4], EP_IN, DEG_OFF[4] + 10000, 32)
    niD2d = _pad3d(ni_dst + DEG_OFF[5], EP_IN, DEG_OFF[5] + 50000, 32)

    d0, d1 = _deg_call(svcS2d, svcD2d, inS2d, inD2d, niS2d, niD2d)

    h_svc = _matmul(feat_svc, _degslice(d0, DEG_OFF[0], NSVC),
                    _degslice(d1, DEG_OFF[0], NSVC), W_svc)
    h_in = _matmul(feat_pod, _degslice(d0, DEG_OFF[2], NPOD),
                   _degslice(d1, DEG_OFF[2], NPOD), W_in)
    h_ni = _matmul(feat_node, _degslice(d0, DEG_OFF[4], NNODE),
                   _degslice(d1, DEG_OFF[4], NNODE), W_ni)

    agg_svc, agg_in, agg_ni = _aggall_call(
        h_svc, h_in, h_ni, svcS1, svcD1, inS1, inD1, niS1, niD1)

    s1 = _reduce([(agg_svc, 0), (agg_svc, 28)],
                 _degslice(d0, DEG_OFF[1], NSVC), _degslice(d1, DEG_OFF[1], NSVC),
                 b_svc.reshape(1, 128), Wl_svc, bl_svc.reshape(1, 128),
                 NSVC)
    s2 = _reduce([(agg_in, 0), (agg_in, 28)],
                 _degslice(d0, DEG_OFF[3], NNODE), _degslice(d1, DEG_OFF[3], NNODE),
                 b_in.reshape(1, 128), Wl_node, bl_node.reshape(1, 128),
                 NNODE)
    s3 = _reduce([(agg_ni, 0)],
                 _degslice(d0, DEG_OFF[5], NPOD), _degslice(d1, DEG_OFF[5], NPOD),
                 b_ni.reshape(1, 128), Wl_pod, bl_pod.reshape(1, 128),
                 NPOD)
    total = (s1[0, 0] + s2[0, 0] + s3[0, 0]) / float((NSVC + NPOD + NNODE)
                                                     * 128)
    return total
